# baseline (speedup 1.0000x reference)
import functools

import jax
import jax.numpy as jnp
from jax.experimental import pallas as pl
from jax.experimental.pallas import tpu as pltpu

K = 10
RADIUS = 0.1
MULTIRES = 10
BIG = 1e30
NSUB = 4
SLAB = 512


def _body(xyz_ref, pcdc_ref, featc_ref, zlo_ref, zhi_ref, out_ref,
          d2_ref, cv_ref, *, n_chunks, chunk):
    x = xyz_ref[0]
    pt = x.shape[0]
    r2 = jnp.float32(RADIUS * RADIUS)

    qzmin = jnp.min(x[:, 2]) - jnp.float32(RADIUS)
    qzmax = jnp.max(x[:, 2]) + jnp.float32(RADIUS)
    below = (zhi_ref[0, 0, :] < qzmin).astype(jnp.int32)
    above = (zlo_ref[0, 0, :] > qzmax).astype(jnp.int32)
    c_lo = jnp.sum(below)
    c_hi = n_chunks - jnp.sum(above)

    def d2_sub(c, q):
        acc = None
        for i in range(3):
            diff = x[:, i : i + 1] - pcdc_ref[0, c, q, i, :][None, :]
            sq = diff * diff
            acc = sq if acc is None else acc + sq
        return acc

    def cmpex(a, b):
        return jnp.minimum(a, b), jnp.maximum(a, b)

    cv_ref[...] = jnp.full((n_chunks, pt, K + 1), BIG, jnp.float32)

    def p1_body(c, carry):
        v = []
        for q in range(NSUB):
            d = d2_sub(c, q)
            masked = jnp.where(d <= r2, d, BIG)
            d2_ref[c, q] = masked
            v.append(masked)
        v[0], v[1] = cmpex(v[0], v[1])
        v[2], v[3] = cmpex(v[2], v[3])
        v[0], v[2] = cmpex(v[0], v[2])
        v[1], v[3] = cmpex(v[1], v[3])
        v[1], v[2] = cmpex(v[1], v[2])
        v0, v1, v2, v3 = v
        vals = []
        for _ in range(K + 1):
            m = jnp.min(v0, axis=-1, keepdims=True)
            vals.append(m)
            hit = v0 <= m
            v0 = jnp.where(hit, v1, v0)
            v1 = jnp.where(hit, v2, v1)
            v2 = jnp.where(hit, v3, v2)
            v3 = jnp.where(hit, BIG, v3)
        cv_ref[c] = jnp.concatenate(vals, axis=-1)
        return carry

    jax.lax.fori_loop(c_lo, c_hi, p1_body, jnp.zeros((), jnp.float32))

    allv = cv_ref[...]
    m = None
    for _ in range(K - 1):
        filt = allv if m is None else jnp.where(allv > m, allv, BIG)
        m = jnp.min(filt, axis=(0, 2), keepdims=True)
    t10 = jnp.min(jnp.where(allv > m, allv, BIG), axis=(0, 2), keepdims=True)
    t11 = jnp.min(jnp.where(allv > t10, allv, BIG), axis=(0, 2), keepdims=True)
    t10, t11 = t10[0], t11[0]
    tau = jnp.minimum(t10 + (t11 - t10) * 0.5, r2)

    nfeat = featc_ref.shape[4]

    def p2_body(c, acc_f):
        for q in range(NSUB):
            mask = jnp.where(d2_ref[c, q] <= tau, 1.0, 0.0)
            fc = featc_ref[0, c, q]
            acc_f = acc_f + jax.lax.dot_general(
                mask.astype(jnp.float32), fc, (((1,), (0,)), ((), ())),
                preferred_element_type=jnp.float32)
        return acc_f

    acc_f = jax.lax.fori_loop(
        c_lo, c_hi, p2_body, jnp.zeros((pt, nfeat), jnp.float32))
    cnt = acc_f[:, nfeat - 1 : nfeat]
    fcd = acc_f[:, : nfeat - 1] / jnp.maximum(cnt, 1.0)

    out_dim = out_ref.shape[2]
    nf = nfeat - 1
    li = jax.lax.broadcasted_iota(jnp.int32, (1, out_dim), 1)
    x0 = jnp.broadcast_to(x[:, 0:1], (pt, out_dim))
    x1 = jnp.broadcast_to(x[:, 1:2], (pt, out_dim))
    x2 = jnp.broadcast_to(x[:, 2:3], (pt, out_dim))
    b0, b1 = nf + 3, nf + 3 + 2 * MULTIRES
    b2 = b1 + 2 * MULTIRES
    xs = jnp.where(li < b1, x0, jnp.where(li < b2, x1, x2))
    xs = jnp.where(li == nf + 1, x1, xs)
    xs = jnp.where(li == nf + 2, x2, xs)
    blk = jnp.where(li < b1, li - b0, jnp.where(li < b2, li - b1, li - b2))
    is_cos = blk >= MULTIRES
    e = jnp.where(is_cos, blk - MULTIRES, blk)
    ftab = jnp.exp2(jnp.where(e < 0, 0, e).astype(jnp.float32))
    arg = xs * ftab
    pe = jnp.where(is_cos, jnp.cos(arg), jnp.sin(arg))
    pe = jnp.where(li < b0, xs, pe)
    fcd_pad = jnp.pad(fcd, ((0, 0), (0, out_dim - nf)))
    out_ref[0] = jnp.where(li < nf, fcd_pad, pe)


def kernel(xyz, pcd, feat):
    b, p, _ = xyz.shape
    n = pcd.shape[1]
    c = feat.shape[2]
    pt = min(256, p)
    chunk = min(SLAB, n)
    n_chunks = n // chunk
    sub = chunk // NSUB
    out_dim = c + 3 + 3 * 2 * MULTIRES

    def bucket_order(z, nbins):
        bb, m = z.shape
        bkt = jnp.clip((z * nbins).astype(jnp.int32), 0, nbins - 1)
        oh = (bkt[:, :, None] == jax.lax.broadcasted_iota(
            jnp.int32, (1, 1, nbins), 2)).astype(jnp.float32)
        csum = jnp.cumsum(oh, axis=1)
        rank = jnp.take_along_axis(csum, bkt[:, :, None], axis=2)[..., 0]
        totals = csum[:, -1, :]
        starts = jnp.cumsum(totals, axis=1) - totals
        pos = (jnp.take_along_axis(starts, bkt, axis=1) + rank - 1.0
               ).astype(jnp.int32)
        bidx = jnp.broadcast_to(
            jnp.arange(bb, dtype=jnp.int32)[:, None], (bb, m))
        iot = jnp.broadcast_to(jnp.arange(m, dtype=jnp.int32)[None], (bb, m))
        inv = jnp.zeros((bb, m), jnp.int32).at[bidx, pos].set(iot)
        return pos, inv

    _, inv_p = bucket_order(pcd[:, :, 2], 32)
    pcd_s = jnp.take_along_axis(pcd, inv_p[:, :, None], axis=1)
    feat_s = jnp.take_along_axis(feat, inv_p[:, :, None], axis=1)
    qpos, inv_q = bucket_order(xyz[:, :, 2], 32)
    xyz_s = jnp.take_along_axis(xyz, inv_q[:, :, None], axis=1)

    zs = pcd_s[:, :, 2].reshape(b, n_chunks, chunk)
    zhi = jax.lax.cummax(zs.max(-1), axis=1).reshape(b, 1, n_chunks)
    zlo = -jax.lax.cummax(
        -zs.min(-1)[:, ::-1], axis=1)[:, ::-1].reshape(b, 1, n_chunks)

    pcd_c = jnp.swapaxes(pcd_s, 1, 2).reshape(b, 3, n_chunks, NSUB, sub)
    pcd_c = jnp.transpose(pcd_c, (0, 2, 3, 1, 4))
    feat_aug = jnp.concatenate(
        [feat_s, jnp.ones((b, n, 1), jnp.float32)], axis=-1)
    feat_c = feat_aug.reshape(b, n_chunks, NSUB, sub, c + 1)

    out_s = pl.pallas_call(
        functools.partial(_body, n_chunks=n_chunks, chunk=chunk),
        grid=(b, p // pt),
        in_specs=[
            pl.BlockSpec((1, pt, 3), lambda bi, pi: (bi, pi, 0)),
            pl.BlockSpec((1, n_chunks, NSUB, 3, sub),
                         lambda bi, pi: (bi, 0, 0, 0, 0)),
            pl.BlockSpec((1, n_chunks, NSUB, sub, c + 1),
                         lambda bi, pi: (bi, 0, 0, 0, 0)),
            pl.BlockSpec((1, 1, n_chunks), lambda bi, pi: (bi, 0, 0)),
            pl.BlockSpec((1, 1, n_chunks), lambda bi, pi: (bi, 0, 0)),
        ],
        out_specs=pl.BlockSpec((1, pt, out_dim), lambda bi, pi: (bi, pi, 0)),
        out_shape=jax.ShapeDtypeStruct((b, p, out_dim), jnp.float32),
        scratch_shapes=[pltpu.VMEM((n_chunks, NSUB, pt, sub), jnp.float32),
                        pltpu.VMEM((n_chunks, pt, K + 1), jnp.float32)],
    )(xyz_s, pcd_c, feat_c, zlo, zhi)

    return jnp.take_along_axis(out_s, qpos[:, :, None], axis=1)


# speedup vs baseline: 1.0531x; 1.0531x over previous
import functools

import jax
import jax.numpy as jnp
from jax.experimental import pallas as pl
from jax.experimental.pallas import tpu as pltpu

K = 10
RADIUS = 0.1
MULTIRES = 10
BIG = 1e30
NSUB = 4
SLAB = 512


def _body(xyz_ref, pcdc_ref, featc_ref, zlo_ref, zhi_ref, out_ref,
          d2_ref, cv_ref, *, n_chunks, chunk):
    x = xyz_ref[0]
    pt = x.shape[0]
    r2 = jnp.float32(RADIUS * RADIUS)

    qzmin = xyz_ref[0, 0, 2] - jnp.float32(RADIUS)
    qzmax = xyz_ref[0, pt - 1, 2] + jnp.float32(RADIUS)
    below = (zhi_ref[0, 0, :] < qzmin).astype(jnp.int32)
    above = (zlo_ref[0, 0, :] > qzmax).astype(jnp.int32)
    c_lo = jnp.sum(below)
    c_hi = n_chunks - jnp.sum(above)

    def d2_sub(c, q):
        acc = None
        for i in range(3):
            diff = x[:, i : i + 1] - pcdc_ref[0, c, q, i, :][None, :]
            sq = diff * diff
            acc = sq if acc is None else acc + sq
        return acc

    def cmpex(a, b):
        return jnp.minimum(a, b), jnp.maximum(a, b)

    cv_ref[...] = jnp.full((n_chunks, pt, K + 1), BIG, jnp.float32)

    def p1_body(c, carry):
        v = []
        for q in range(NSUB):
            d = d2_sub(c, q)
            masked = jnp.where(d <= r2, d, BIG)
            d2_ref[c, q] = masked
            v.append(masked)
        v[0], v[1] = cmpex(v[0], v[1])
        v[2], v[3] = cmpex(v[2], v[3])
        v[0], v[2] = cmpex(v[0], v[2])
        v[1], v[3] = cmpex(v[1], v[3])
        v[1], v[2] = cmpex(v[1], v[2])
        v0, v1, v2, v3 = v
        vals = []
        for _ in range(K + 1):
            m = jnp.min(v0, axis=-1, keepdims=True)
            vals.append(m)
            hit = v0 <= m
            v0 = jnp.where(hit, v1, v0)
            v1 = jnp.where(hit, v2, v1)
            v2 = jnp.where(hit, v3, v2)
            v3 = jnp.where(hit, BIG, v3)
        cv_ref[c] = jnp.concatenate(vals, axis=-1)
        return carry

    jax.lax.fori_loop(c_lo, c_hi, p1_body, jnp.zeros((), jnp.float32))

    allv = cv_ref[...]
    m = None
    for _ in range(K - 1):
        filt = allv if m is None else jnp.where(allv > m, allv, BIG)
        m = jnp.min(filt, axis=(0, 2), keepdims=True)
    t10 = jnp.min(jnp.where(allv > m, allv, BIG), axis=(0, 2), keepdims=True)
    t11 = jnp.min(jnp.where(allv > t10, allv, BIG), axis=(0, 2), keepdims=True)
    t10, t11 = t10[0], t11[0]
    tau = jnp.minimum(t10 + (t11 - t10) * 0.5, r2)

    nfeat = featc_ref.shape[4]

    def p2_body(c, acc_f):
        for q in range(NSUB):
            mask = jnp.where(d2_ref[c, q] <= tau, 1.0, 0.0)
            fc = featc_ref[0, c, q]
            acc_f = acc_f + jax.lax.dot_general(
                mask.astype(jnp.bfloat16), fc, (((1,), (0,)), ((), ())),
                preferred_element_type=jnp.float32)
        return acc_f

    acc_f = jax.lax.fori_loop(
        c_lo, c_hi, p2_body, jnp.zeros((pt, nfeat), jnp.float32))
    cnt = acc_f[:, nfeat - 1 : nfeat]
    fcd = acc_f[:, : nfeat - 1] / jnp.maximum(cnt, 1.0)

    out_dim = out_ref.shape[2]
    nf = nfeat - 1
    li = jax.lax.broadcasted_iota(jnp.int32, (1, out_dim), 1)
    x0 = jnp.broadcast_to(x[:, 0:1], (pt, out_dim))
    x1 = jnp.broadcast_to(x[:, 1:2], (pt, out_dim))
    x2 = jnp.broadcast_to(x[:, 2:3], (pt, out_dim))
    b0, b1 = nf + 3, nf + 3 + 2 * MULTIRES
    b2 = b1 + 2 * MULTIRES
    xs = jnp.where(li < b1, x0, jnp.where(li < b2, x1, x2))
    xs = jnp.where(li == nf + 1, x1, xs)
    xs = jnp.where(li == nf + 2, x2, xs)
    blk = jnp.where(li < b1, li - b0, jnp.where(li < b2, li - b1, li - b2))
    is_cos = blk >= MULTIRES
    e = jnp.where(is_cos, blk - MULTIRES, blk)
    ftab = jnp.exp2(jnp.where(e < 0, 0, e).astype(jnp.float32))
    arg = xs * ftab
    pe = jnp.where(is_cos, jnp.cos(arg), jnp.sin(arg))
    pe = jnp.where(li < b0, xs, pe)
    fcd_pad = jnp.pad(fcd, ((0, 0), (0, out_dim - nf)))
    out_ref[0] = jnp.where(li < nf, fcd_pad, pe)


def kernel(xyz, pcd, feat):
    b, p, _ = xyz.shape
    n = pcd.shape[1]
    c = feat.shape[2]
    pt = min(512, p)
    chunk = min(SLAB, n)
    n_chunks = n // chunk
    sub = chunk // NSUB
    out_dim = c + 3 + 3 * 2 * MULTIRES

    pperm = jnp.argsort(pcd[:, :, 2], axis=1)
    pcd_s = jnp.take_along_axis(pcd, pperm[:, :, None], axis=1)
    feat_s = jnp.take_along_axis(
        feat.astype(jnp.bfloat16), pperm[:, :, None], axis=1)
    qperm = jnp.argsort(xyz[:, :, 2], axis=1)
    xyz_s = jnp.take_along_axis(xyz, qperm[:, :, None], axis=1)

    zlo = pcd_s[:, ::chunk, 2].reshape(b, 1, n_chunks)
    zhi = pcd_s[:, chunk - 1 :: chunk, 2].reshape(b, 1, n_chunks)

    pcd_c = jnp.swapaxes(pcd_s, 1, 2).reshape(b, 3, n_chunks, NSUB, sub)
    pcd_c = jnp.transpose(pcd_c, (0, 2, 3, 1, 4))
    feat_aug = jnp.concatenate(
        [feat_s, jnp.ones((b, n, 1), jnp.bfloat16)], axis=-1)
    feat_c = feat_aug.reshape(b, n_chunks, NSUB, sub, c + 1)

    out_s = pl.pallas_call(
        functools.partial(_body, n_chunks=n_chunks, chunk=chunk),
        grid=(b, p // pt),
        in_specs=[
            pl.BlockSpec((1, pt, 3), lambda bi, pi: (bi, pi, 0)),
            pl.BlockSpec((1, n_chunks, NSUB, 3, sub),
                         lambda bi, pi: (bi, 0, 0, 0, 0)),
            pl.BlockSpec((1, n_chunks, NSUB, sub, c + 1),
                         lambda bi, pi: (bi, 0, 0, 0, 0)),
            pl.BlockSpec((1, 1, n_chunks), lambda bi, pi: (bi, 0, 0)),
            pl.BlockSpec((1, 1, n_chunks), lambda bi, pi: (bi, 0, 0)),
        ],
        out_specs=pl.BlockSpec((1, pt, out_dim), lambda bi, pi: (bi, pi, 0)),
        out_shape=jax.ShapeDtypeStruct((b, p, out_dim), jnp.float32),
        scratch_shapes=[pltpu.VMEM((n_chunks, NSUB, pt, sub), jnp.float32),
                        pltpu.VMEM((n_chunks, pt, K + 1), jnp.float32)],
    )(xyz_s, pcd_c, feat_c, zlo, zhi)

    inv = jnp.argsort(qperm, axis=1)
    return jnp.take_along_axis(out_s, inv[:, :, None], axis=1)


# speedup vs baseline: 1.7065x; 1.6205x over previous
import functools

import jax
import jax.numpy as jnp
from jax.experimental import pallas as pl
from jax.experimental.pallas import tpu as pltpu

K = 10
RADIUS = 0.1
MULTIRES = 10
BIG = 1e30
NSUB = 4
SLAB = 512


def _body(xyz_ref, pcdc_ref, featc_ref, zlo_ref, zhi_ref, out_ref,
          d2_ref, cv_ref, *, n_chunks, chunk):
    x = xyz_ref[0]
    pt = x.shape[0]
    r2 = jnp.float32(RADIUS * RADIUS)

    qzmin = xyz_ref[0, 0, 2] - jnp.float32(RADIUS)
    qzmax = xyz_ref[0, pt - 1, 2] + jnp.float32(RADIUS)
    below = (zhi_ref[0, 0, :] < qzmin).astype(jnp.int32)
    above = (zlo_ref[0, 0, :] > qzmax).astype(jnp.int32)
    c_lo = jnp.sum(below)
    c_hi = n_chunks - jnp.sum(above)

    def d2_sub(c, q):
        acc = None
        for i in range(3):
            diff = x[:, i : i + 1] - pcdc_ref[0, c, q, i, :][None, :]
            sq = diff * diff
            acc = sq if acc is None else acc + sq
        return acc

    def cmpex(a, b):
        return jnp.minimum(a, b), jnp.maximum(a, b)

    cv_ref[...] = jnp.full((n_chunks, pt, K + 1), BIG, jnp.float32)

    def p1_body(c, carry):
        v = []
        for q in range(NSUB):
            d = d2_sub(c, q)
            masked = jnp.where(d <= r2, d, BIG)
            d2_ref[c, q] = masked
            v.append(masked)
        v[0], v[1] = cmpex(v[0], v[1])
        v[2], v[3] = cmpex(v[2], v[3])
        v[0], v[2] = cmpex(v[0], v[2])
        v[1], v[3] = cmpex(v[1], v[3])
        v[1], v[2] = cmpex(v[1], v[2])
        v0, v1, v2, v3 = v
        vals = []
        for _ in range(K + 1):
            m = jnp.min(v0, axis=-1, keepdims=True)
            vals.append(m)
            hit = v0 <= m
            v0 = jnp.where(hit, v1, v0)
            v1 = jnp.where(hit, v2, v1)
            v2 = jnp.where(hit, v3, v2)
            v3 = jnp.where(hit, BIG, v3)
        cv_ref[c] = jnp.concatenate(vals, axis=-1)
        return carry

    jax.lax.fori_loop(c_lo, c_hi, p1_body, jnp.zeros((), jnp.float32))

    allv = cv_ref[...]
    m = None
    for _ in range(K - 1):
        filt = allv if m is None else jnp.where(allv > m, allv, BIG)
        m = jnp.min(filt, axis=(0, 2), keepdims=True)
    t10 = jnp.min(jnp.where(allv > m, allv, BIG), axis=(0, 2), keepdims=True)
    t11 = jnp.min(jnp.where(allv > t10, allv, BIG), axis=(0, 2), keepdims=True)
    t10, t11 = t10[0], t11[0]
    tau = jnp.minimum(t10 + (t11 - t10) * 0.5, r2)

    nfeat = featc_ref.shape[4]

    def p2_body(c, acc_f):
        for q in range(NSUB):
            mask = jnp.where(d2_ref[c, q] <= tau, 1.0, 0.0)
            fc = featc_ref[0, c, q]
            acc_f = acc_f + jax.lax.dot_general(
                mask.astype(jnp.float32), fc, (((1,), (0,)), ((), ())),
                preferred_element_type=jnp.float32)
        return acc_f

    acc_f = jax.lax.fori_loop(
        c_lo, c_hi, p2_body, jnp.zeros((pt, nfeat), jnp.float32))
    cnt = acc_f[:, nfeat - 1 : nfeat]
    fcd = acc_f[:, : nfeat - 1] / jnp.maximum(cnt, 1.0)

    out_dim = out_ref.shape[2]
    nf = nfeat - 1
    li = jax.lax.broadcasted_iota(jnp.int32, (1, out_dim), 1)
    x0 = jnp.broadcast_to(x[:, 0:1], (pt, out_dim))
    x1 = jnp.broadcast_to(x[:, 1:2], (pt, out_dim))
    x2 = jnp.broadcast_to(x[:, 2:3], (pt, out_dim))
    b0, b1 = nf + 3, nf + 3 + 2 * MULTIRES
    b2 = b1 + 2 * MULTIRES
    xs = jnp.where(li < b1, x0, jnp.where(li < b2, x1, x2))
    xs = jnp.where(li == nf + 1, x1, xs)
    xs = jnp.where(li == nf + 2, x2, xs)
    blk = jnp.where(li < b1, li - b0, jnp.where(li < b2, li - b1, li - b2))
    is_cos = blk >= MULTIRES
    e = jnp.where(is_cos, blk - MULTIRES, blk)
    ftab = jnp.exp2(jnp.where(e < 0, 0, e).astype(jnp.float32))
    arg = xs * ftab
    pe = jnp.where(is_cos, jnp.cos(arg), jnp.sin(arg))
    pe = jnp.where(li < b0, xs, pe)
    fcd_pad = jnp.pad(fcd, ((0, 0), (0, out_dim - nf)))
    out_ref[0] = jnp.where(li < nf, fcd_pad, pe)


def kernel(xyz, pcd, feat):
    b, p, _ = xyz.shape
    n = pcd.shape[1]
    c = feat.shape[2]
    pt = min(256, p)
    chunk = min(SLAB, n)
    n_chunks = n // chunk
    sub = chunk // NSUB
    out_dim = c + 3 + 3 * 2 * MULTIRES

    pperm = jnp.argsort(pcd[:, :, 2], axis=1)
    pcd_s = jnp.take_along_axis(pcd, pperm[:, :, None], axis=1)
    feat_s = jnp.take_along_axis(feat, pperm[:, :, None], axis=1)
    qperm = jnp.argsort(xyz[:, :, 2], axis=1)
    xyz_s = jnp.take_along_axis(xyz, qperm[:, :, None], axis=1)

    zlo = pcd_s[:, ::chunk, 2].reshape(b, 1, n_chunks)
    zhi = pcd_s[:, chunk - 1 :: chunk, 2].reshape(b, 1, n_chunks)

    pcd_c = jnp.swapaxes(pcd_s, 1, 2).reshape(b, 3, n_chunks, NSUB, sub)
    pcd_c = jnp.transpose(pcd_c, (0, 2, 3, 1, 4))
    feat_aug = jnp.concatenate(
        [feat_s, jnp.ones((b, n, 1), jnp.float32)], axis=-1)
    feat_c = feat_aug.reshape(b, n_chunks, NSUB, sub, c + 1)

    out_s = pl.pallas_call(
        functools.partial(_body, n_chunks=n_chunks, chunk=chunk),
        grid=(b, p // pt),
        in_specs=[
            pl.BlockSpec((1, pt, 3), lambda bi, pi: (bi, pi, 0)),
            pl.BlockSpec((1, n_chunks, NSUB, 3, sub),
                         lambda bi, pi: (bi, 0, 0, 0, 0)),
            pl.BlockSpec((1, n_chunks, NSUB, sub, c + 1),
                         lambda bi, pi: (bi, 0, 0, 0, 0)),
            pl.BlockSpec((1, 1, n_chunks), lambda bi, pi: (bi, 0, 0)),
            pl.BlockSpec((1, 1, n_chunks), lambda bi, pi: (bi, 0, 0)),
        ],
        out_specs=pl.BlockSpec((1, pt, out_dim), lambda bi, pi: (bi, pi, 0)),
        out_shape=jax.ShapeDtypeStruct((b, p, out_dim), jnp.float32),
        scratch_shapes=[pltpu.VMEM((n_chunks, NSUB, pt, sub), jnp.float32),
                        pltpu.VMEM((n_chunks, pt, K + 1), jnp.float32)],
    )(xyz_s, pcd_c, feat_c, zlo, zhi)

    inv = jnp.argsort(qperm, axis=1)
    return jnp.take_along_axis(out_s, inv[:, :, None], axis=1)
